# baseline (speedup 1.0000x reference)
import functools

import jax
import jax.numpy as jnp
from jax import lax
from jax.experimental import pallas as pl

_N_BLK = 32768


def _fused_kernel(batch_ref, h_ref, w1_t_ref, ret_feat_ref, w_ret_t_ref,
                  b_ret_ref, w2_t_ref, b_lin_ref, out_ref, *, nb, b):
    rm = jnp.mean(ret_feat_ref[...], axis=1)
    rp = jnp.dot(rm, w_ret_t_ref[...],
                 preferred_element_type=jnp.float32) + b_ret_ref[...]
    ret2 = jnp.dot(rp, w2_t_ref[...],
                   preferred_element_type=jnp.float32) + b_lin_ref[...]
    idx = batch_ref[0, :]
    oh = (idx[:, None] == lax.broadcasted_iota(jnp.int32, (nb, b), 1)
          ).astype(jnp.float32)
    out_ref[...] = (
        jnp.dot(h_ref[...], w1_t_ref[...],
                preferred_element_type=jnp.float32)
        + jnp.dot(oh, ret2, preferred_element_type=jnp.float32))


def kernel(h, ret_feat, batch, W_ret, b_ret, W_lin, b_lin):
    n, h_dim = h.shape
    bsz, r, ret_dim = ret_feat.shape
    w1_t = W_lin[:, :h_dim].T
    w2_t = W_lin[:, h_dim:].T

    nblk = _N_BLK
    grid = n // nblk
    batch3 = batch.reshape(grid, 1, nblk)
    zero = lambda i: (0, 0)
    out = pl.pallas_call(
        functools.partial(_fused_kernel, nb=nblk, b=bsz),
        grid=(grid,),
        in_specs=[
            pl.BlockSpec((None, 1, nblk), lambda i: (i, 0, 0)),
            pl.BlockSpec((nblk, h_dim), lambda i: (i, 0)),
            pl.BlockSpec((h_dim, h_dim), zero),
            pl.BlockSpec((bsz, r, ret_dim), lambda i: (0, 0, 0)),
            pl.BlockSpec((h_dim, h_dim), zero),
            pl.BlockSpec((1, h_dim), zero),
            pl.BlockSpec((h_dim, h_dim), zero),
            pl.BlockSpec((1, h_dim), zero),
        ],
        out_specs=pl.BlockSpec((nblk, h_dim), lambda i: (i, 0)),
        out_shape=jax.ShapeDtypeStruct((n, h_dim), jnp.float32),
    )(batch3, h, w1_t, ret_feat, W_ret.T, b_ret.reshape(1, h_dim), w2_t,
      b_lin.reshape(1, h_dim))
    return out
